# baseline (speedup 1.0000x reference)
import functools

import jax
import jax.numpy as jnp
from jax import lax
from jax.experimental import pallas as pl
from jax.experimental.pallas import tpu as pltpu
from jax.experimental.pallas import tpu_sc as plsc

_N_USERS = 50000
_N_ITEMS = 25000
_N_ENT = 50000
_D = 64
_E = 800000

_NC = 2
_NS = 16
_L = 16
_NCH = _D // _L

_K = 128
_E_PAD = 802816
_E_T = _E_PAD // _NS
_BLK_T = _E_T // _K

_N_STAGE = 14
_E_S = _E_T // _N_STAGE
_NBLK_S = _E_S // _K
_NBUF = 4
_NGRP = _NBLK_S // _NBUF
_WB = 48

_NP = 75264


def _make_fused():
    acc_rows = _NP + 128
    z_per_tile = acc_rows // _NS
    w_per_tile = _NP // _NS
    mesh = plsc.VectorSubcoreMesh(core_axis_name="c", subcore_axis_name="s")

    scratch = [
        pltpu.VMEM((2, _E_S), jnp.int32),
        pltpu.VMEM((2, _NBLK_S, _K), jnp.int32),
        pltpu.VMEM((2, _E_S), jnp.float32),
        pltpu.VMEM((_NBUF, _K, _L), jnp.float32),
        pltpu.VMEM((_NBUF, _K, _L), jnp.float32),
        pltpu.VMEM((_WB, _L), jnp.float32),
        pltpu.VMEM((2, _WB, _L), jnp.float32),
        pltpu.VMEM((2, _WB, _L), jnp.float32),
        pltpu.VMEM((2, _WB, _L), jnp.float32),
        pltpu.VMEM_SHARED((acc_rows, _L), jnp.float32),
    ] + [pltpu.SemaphoreType.DMA] * (2 * _NBUF + 7)

    @functools.partial(
        pl.kernel,
        mesh=mesh,
        out_type=(jax.ShapeDtypeStruct((_NP * _NCH, _L), jnp.float32),
                  jax.ShapeDtypeStruct((_NP * _NCH, _L), jnp.float32)),
        scratch_types=scratch,
        compiler_params=pltpu.CompilerParams(
            use_tc_tiling_on_sc=False, needs_layout_passes=False),
    )
    def fused(*refs):
        (tab, colb, rowb, wb, zsrc, h1out, fin,
         colbuf, rowbuf, wvbuf, data, sdata, zstage, wbuf, xbuf, hbuf,
         acc, *sems) = refs
        gsem = sems[:_NBUF]
        ssem = sems[_NBUF:2 * _NBUF]
        psem = sems[2 * _NBUF]
        wsem = sems[2 * _NBUF + 1:2 * _NBUF + 3]
        xsem = sems[2 * _NBUF + 3:2 * _NBUF + 5]
        hsem = sems[2 * _NBUF + 5:2 * _NBUF + 7]
        cid = lax.axis_index("c")
        sid = lax.axis_index("s")
        e0 = sid * _E_T
        eblk0 = sid * _BLK_T
        o0 = sid * w_per_tile
        pltpu.sync_copy(zsrc, zstage)

        def prefetch(stage, slot):
            sbase = e0 + stage * _E_S
            sblk = eblk0 + stage * _NBLK_S
            pltpu.async_copy(colb.at[pl.ds(sbase, _E_S)],
                             colbuf.at[slot], psem)
            pltpu.async_copy(rowb.at[pl.ds(sblk, _NBLK_S)],
                             rowbuf.at[slot], psem)
            pltpu.async_copy(wb.at[pl.ds(sbase, _E_S)],
                             wvbuf.at[slot], psem)

        def prefetch_wait(slot):
            pltpu.make_async_copy(colb.at[pl.ds(0, _E_S)],
                                  colbuf.at[slot], psem).wait()
            pltpu.make_async_copy(rowb.at[pl.ds(0, _NBLK_S)],
                                  rowbuf.at[slot], psem).wait()
            pltpu.make_async_copy(wb.at[pl.ds(0, _E_S)],
                                  wvbuf.at[slot], psem).wait()

        def scatter_phase(src_tab, cbase):
            prefetch(0, 0)

            @pl.loop(0, _N_STAGE)
            def _stage(stage):
                slot = jnp.bitwise_and(stage, 1)
                prefetch_wait(slot)

                @pl.when(stage < _N_STAGE - 1)
                def _():
                    prefetch(stage + 1, 1 - slot)

                gidx_s = colbuf.at[slot]
                row_s = rowbuf.at[slot]
                wv_s = wvbuf.at[slot]

                @pl.loop(0, _E_S // _L, unroll=4)
                def _bi(j):
                    o = j * _L
                    gidx_s[pl.ds(o, _L)] = gidx_s[pl.ds(o, _L)] + cbase

                for b in range(_NBUF):
                    pltpu.async_copy(
                        src_tab.at[gidx_s.at[pl.ds(b * _K, _K)]],
                        data.at[b], gsem[b])

                @pl.loop(0, _NGRP)
                def _grp(g):
                    for b in range(_NBUF):
                        blk = g * _NBUF + b
                        pltpu.make_async_copy(
                            src_tab.at[gidx_s.at[pl.ds(blk * _K, _K)]],
                            data.at[b], gsem[b]).wait()
                        db = data.at[b]
                        sb = sdata.at[b]

                        @pl.when(g > 0)
                        def _():
                            pltpu.make_async_copy(
                                sb, acc.at[row_s.at[blk - _NBUF]],
                                ssem[b]).wait()

                        @pl.loop(0, _K // _L)
                        def _wg(j):
                            wv16 = wv_s[pl.ds(blk * _K + j * _L, _L)]
                            for i in range(_L):
                                e = j * _L + i
                                wvec = wv16[jnp.full((_L,), i, jnp.int32)]
                                sb[e] = db[e] * wvec
                        pltpu.async_copy(sb, acc.at[row_s.at[blk]],
                                         ssem[b], add=True)

                        @pl.when(blk + _NBUF < _NBLK_S)
                        def _():
                            pltpu.async_copy(
                                src_tab.at[gidx_s.at[
                                    pl.ds((blk + _NBUF) * _K, _K)]],
                                data.at[b], gsem[b])

                for b in range(_NBUF):
                    lastblk = (_NGRP - 1) * _NBUF + b
                    pltpu.make_async_copy(
                        sdata.at[b], acc.at[row_s.at[lastblk]],
                        ssem[b]).wait()

        def wb_plain(dst, cbase):
            def fill(i, b):
                off = o0 + i * _WB
                pltpu.sync_copy(acc.at[pl.ds(off, _WB)], wbuf.at[b])
                pltpu.sync_copy(zstage, acc.at[pl.ds(off, _WB)])
                pltpu.async_copy(wbuf.at[b],
                                 dst.at[pl.ds(cbase + off, _WB)], wsem[b])

            def wait(b):
                pltpu.make_async_copy(wbuf.at[b],
                                      dst.at[pl.ds(cbase + o0, _WB)],
                                      wsem[b]).wait()

            nblk = w_per_tile // _WB
            for b in range(2):
                fill(b, b)

            @pl.loop(0, nblk // 2 - 1)
            def _wb(i):
                for b in range(2):
                    wait(b)
                    fill(2 + i * 2 + b, b)
            for b in range(2):
                wait(b)

        def wb_combine(dst, xr, h1r, cbase, p):
            def readxh(i, b):
                off = cbase + o0 + i * _WB
                pltpu.async_copy(xr.at[pl.ds(off, _WB)], xbuf.at[b], xsem[b])
                pltpu.async_copy(h1r.at[pl.ds(off, _WB)], hbuf.at[b], hsem[b])

            def slot(i, b, do_wait, do_prefetch):
                off = o0 + i * _WB
                pltpu.make_async_copy(xr.at[pl.ds(cbase + off, _WB)],
                                      xbuf.at[b], xsem[b]).wait()
                pltpu.make_async_copy(h1r.at[pl.ds(cbase + off, _WB)],
                                      hbuf.at[b], hsem[b]).wait()
                if do_wait:
                    pltpu.make_async_copy(
                        wbuf.at[b], dst.at[pl.ds(cbase + o0, _WB)],
                        wsem[b]).wait()
                pltpu.sync_copy(acc.at[pl.ds(off, _WB)], wbuf.at[b])

                @pl.when(p == 0)
                def _():
                    pltpu.sync_copy(zstage, acc.at[pl.ds(off, _WB)])

                wbb, xbb, hbb = wbuf.at[b], xbuf.at[b], hbuf.at[b]

                @pl.loop(0, _WB, unroll=4)
                def _cmb(r):
                    wbb[r] = (wbb[r] + xbb[r] + hbb[r]) * (1.0 / 3.0)
                pltpu.async_copy(wbuf.at[b],
                                 dst.at[pl.ds(cbase + off, _WB)], wsem[b])
                if do_prefetch:
                    readxh(i + 2, b)

            nblk = w_per_tile // _WB
            for b in range(2):
                readxh(b, b)
            for b in range(2):
                slot(b, b, False, True)

            @pl.loop(0, nblk // 2 - 2)
            def _wc(i):
                for b in range(2):
                    slot(2 + i * 2 + b, b, True, True)
            for b in range(2):
                slot(nblk - 2 + b, b, True, False)
            for b in range(2):
                pltpu.make_async_copy(
                    wbuf.at[b], dst.at[pl.ds(cbase + o0, _WB)],
                    wsem[b]).wait()

        @pl.loop(0, 2)
        def _pass(p):
            chunk = cid * 2 + p
            cbase = chunk * _NP

            @pl.when(p == 0)
            def _():
                r0 = sid * z_per_tile
                nz_full, nz_rem = z_per_tile // _WB, z_per_tile % _WB

                @pl.loop(0, nz_full)
                def _z(i):
                    pltpu.sync_copy(zstage, acc.at[pl.ds(r0 + i * _WB, _WB)])
                if nz_rem:
                    pltpu.sync_copy(zstage.at[pl.ds(0, nz_rem)],
                                    acc.at[pl.ds(r0 + nz_full * _WB, nz_rem)])
            plsc.subcore_barrier()

            scatter_phase(tab, cbase)
            plsc.subcore_barrier()
            wb_plain(h1out, cbase)
            plsc.subcore_barrier()
            scatter_phase(h1out, cbase)
            plsc.subcore_barrier()
            wb_combine(fin, tab, h1out, cbase, p)
            plsc.subcore_barrier()

    return fused


def _pad_edges(row, col, w):
    pad = _E_PAD - _E
    row_p = jnp.concatenate(
        [row, jnp.full((pad,), _NP, jnp.int32)]).reshape(_E_PAD // _K, _K)
    col_p = jnp.concatenate([col, jnp.zeros((pad,), jnp.int32)])
    w_p = jnp.concatenate([w, jnp.zeros((pad,), jnp.float32)])
    return row_p, col_p, w_p


def _to_chunk_major(x):
    return x.reshape(x.shape[0], _NCH, _L).transpose(1, 0, 2)


def kernel(entity_emb, user_emb, kg_edge_index, kg_edge_weight,
           ui_edge_index, ui_edge_weight):
    f32 = jnp.float32
    zsrc = jnp.zeros((_WB, _L), f32)
    fused = _make_fused()

    krow_p, kcol_p, kw_p = _pad_edges(
        kg_edge_index[0], kg_edge_index[1], kg_edge_weight)
    xe_cm = jnp.pad(_to_chunk_major(entity_emb),
                    ((0, 0), (0, _NP - _N_ENT), (0, 0)))
    _, ent_fin = fused(xe_cm.reshape(_NP * _NCH, _L),
                       kcol_p, krow_p, kw_p, zsrc)
    ent_cm = ent_fin.reshape(_NCH, _NP, _L)

    urow_p, ucol_p, uw_p = _pad_edges(
        ui_edge_index[0], ui_edge_index[1], ui_edge_weight)
    ui_cm = jnp.concatenate(
        [_to_chunk_major(user_emb), ent_cm[:, :_N_ITEMS],
         jnp.zeros((_NCH, _NP - _N_USERS - _N_ITEMS, _L), f32)], axis=1)
    _, ui_fin = fused(ui_cm.reshape(_NP * _NCH, _L),
                      ucol_p, urow_p, uw_p, zsrc)

    entity_out = ent_cm[:, :_N_ENT].transpose(1, 0, 2).reshape(_N_ENT, _D)
    user_out = (ui_fin.reshape(_NCH, _NP, _L)[:, :_N_USERS]
                .transpose(1, 0, 2).reshape(_N_USERS, _D))
    return (user_out, entity_out)
